# Initial kernel scaffold; baseline (speedup 1.0000x reference)
import jax
import jax.numpy as jnp
from jax.experimental import pallas as pl


def kernel(semantic_logits, center_heatmap, offset_map):
    raise NotImplementedError("write your pallas kernel here")


# speedup vs baseline: 12.3173x; 12.3173x over previous
import functools

import jax
import jax.numpy as jnp
from jax import lax
from jax.experimental import pallas as pl
from jax.experimental.pallas import tpu as pltpu

_NUM_CLASSES = 19
_THING_LO = 11
_THING_HI = 18
_CENTER_THRESHOLD = 0.1
_NMS_PAD = 3
_TOP_K = 200
_H = 384
_W = 384
_NEG_INF = float("-inf")


def _argmax_body(sem_ref, out_ref):
    x = sem_ref[0]
    best = x[0]
    bidx = jnp.zeros(best.shape, jnp.int32)
    for c in range(1, _NUM_CLASSES):
        better = x[c] > best
        best = jnp.where(better, x[c], best)
        bidx = jnp.where(better, c, bidx)
    out_ref[...] = bidx


def _main_body(heat_ref, off_ref, sempred_ref, pan_ref,
               scores_ref, cy_s, cx_s, val_s, cnt_s, cls_s, acc_s, iid_s):
    heat = heat_ref[...]

    ninf_rows = jnp.full((_NMS_PAD, _W), _NEG_INF, jnp.float32)
    hp = jnp.concatenate([ninf_rows, heat, ninf_rows], axis=0)
    rm = hp[0:_H, :]
    for d in range(1, 2 * _NMS_PAD + 1):
        rm = jnp.maximum(rm, hp[d:d + _H, :])
    ninf_cols = jnp.full((_H, _NMS_PAD), _NEG_INF, jnp.float32)
    cp = jnp.concatenate([ninf_cols, rm, ninf_cols], axis=1)
    nms = cp[:, 0:_W]
    for d in range(1, 2 * _NMS_PAD + 1):
        nms = jnp.maximum(nms, cp[:, d:d + _W])

    cmask = (heat > _CENTER_THRESHOLD) & (heat == nms)
    scores_ref[...] = jnp.where(cmask, heat, _NEG_INF)

    ri = lax.broadcasted_iota(jnp.int32, (_H, _W), 0)
    ci = lax.broadcasted_iota(jnp.int32, (_H, _W), 1)
    fidx = ri * _W + ci

    def topk_body(k, carry):
        s = scores_ref[...]
        m = jnp.max(s)
        idx = jnp.min(jnp.where(s == m, fidx, jnp.int32(1 << 30)))
        scores_ref[...] = jnp.where(fidx == idx, _NEG_INF, s)
        val_s[k] = m
        cy_s[k] = idx // _W
        cx_s[k] = idx % _W
        return carry

    lax.fori_loop(0, _TOP_K, topk_body, 0)

    sy = ri.astype(jnp.float32) + off_ref[0]
    sx = ci.astype(jnp.float32) + off_ref[1]
    sempred = sempred_ref[...]

    def passa_body(k, carry):
        cyf = cy_s[k].astype(jnp.float32)
        cxf = cx_s[k].astype(jnp.float32)
        dy = sy - cyf
        dx = sx - cxf
        in_disk = dy * dy + dx * dx < 1.0
        cnt_s[k] = jnp.sum(in_disk.astype(jnp.int32))
        eq = (ri == cy_s[k]) & (ci == cx_s[k])
        cls_s[k] = jnp.max(jnp.where(eq, sempred, jnp.int32(-1)))
        return carry

    lax.fori_loop(0, _TOP_K, passa_body, 0)

    def accept_body(k, next_id):
        v = val_s[k]
        valid = (v > _NEG_INF) & (v < jnp.inf)
        is_thing = (cls_s[k] >= _THING_LO) & (cls_s[k] <= _THING_HI)
        a = valid & (cnt_s[k] >= 32) & is_thing
        acc_s[k] = a.astype(jnp.int32)
        iid_s[k] = next_id
        return next_id + a.astype(jnp.int32)

    lax.fori_loop(0, _TOP_K, accept_body, jnp.int32(1000))

    pan_ref[...] = sempred

    def passb_body(k, carry):
        @pl.when(acc_s[k] == 1)
        def _():
            cyf = cy_s[k].astype(jnp.float32)
            cxf = cx_s[k].astype(jnp.float32)
            dy = sy - cyf
            dx = sx - cxf
            in_disk = dy * dy + dx * dx < 1.0
            pan_ref[...] = jnp.where(in_disk, iid_s[k], pan_ref[...])
        return carry

    lax.fori_loop(0, _TOP_K, passb_body, 0)


@jax.jit
def kernel(semantic_logits, center_heatmap, offset_map):
    sem = semantic_logits[0]
    heat = center_heatmap[0, 0]
    off = offset_map[0]

    n_blk = 8
    rows = _H // n_blk
    sempred = pl.pallas_call(
        _argmax_body,
        grid=(n_blk,),
        in_specs=[pl.BlockSpec((1, _NUM_CLASSES, rows, _W),
                               lambda i: (0, 0, i, 0))],
        out_specs=pl.BlockSpec((rows, _W), lambda i: (i, 0)),
        out_shape=jax.ShapeDtypeStruct((_H, _W), jnp.int32),
    )(sem[None])

    pan = pl.pallas_call(
        _main_body,
        out_shape=jax.ShapeDtypeStruct((_H, _W), jnp.int32),
        scratch_shapes=[
            pltpu.VMEM((_H, _W), jnp.float32),
            pltpu.SMEM((_TOP_K,), jnp.int32),
            pltpu.SMEM((_TOP_K,), jnp.int32),
            pltpu.SMEM((_TOP_K,), jnp.float32),
            pltpu.SMEM((_TOP_K,), jnp.int32),
            pltpu.SMEM((_TOP_K,), jnp.int32),
            pltpu.SMEM((_TOP_K,), jnp.int32),
            pltpu.SMEM((_TOP_K,), jnp.int32),
        ],
    )(heat, off, sempred)

    return (semantic_logits, center_heatmap, offset_map, pan[None])


# speedup vs baseline: 19.9135x; 1.6167x over previous
import jax
import jax.numpy as jnp
from jax import lax
from jax.experimental import pallas as pl
from jax.experimental.pallas import tpu as pltpu

_NUM_CLASSES = 19
_THING_LO = 11
_THING_HI = 18
_CENTER_THRESHOLD = 0.1
_NMS_PAD = 3
_TOP_K = 200
_H = 384
_W = 384
_NEG_INF = float("-inf")
_SLAB = 8
_N_SLABS = _H // _SLAB
_WIN = 40
_MAX_OFF = 14.0


def _argmax_body(sem_ref, out_ref):
    x = sem_ref[0]
    best = x[0]
    bidx = jnp.zeros(best.shape, jnp.int32)
    for c in range(1, _NUM_CLASSES):
        better = x[c] > best
        best = jnp.where(better, x[c], best)
        bidx = jnp.where(better, c, bidx)
    out_ref[...] = bidx


def _main_body(heat_ref, off_ref, sempred_ref, pan_ref,
               scores_ref, sy_ref, sx_ref, cnt_s):
    heat = heat_ref[...]

    ninf_rows = jnp.full((_NMS_PAD, _W), _NEG_INF, jnp.float32)
    hp = jnp.concatenate([ninf_rows, heat, ninf_rows], axis=0)
    rm = hp[0:_H, :]
    for d in range(1, 2 * _NMS_PAD + 1):
        rm = jnp.maximum(rm, hp[d:d + _H, :])
    ninf_cols = jnp.full((_H, _NMS_PAD), _NEG_INF, jnp.float32)
    cp = jnp.concatenate([ninf_cols, rm, ninf_cols], axis=1)
    nms = cp[:, 0:_W]
    for d in range(1, 2 * _NMS_PAD + 1):
        nms = jnp.maximum(nms, cp[:, d:d + _W])

    cmask = (heat > _CENTER_THRESHOLD) & (heat == nms)
    scores = jnp.where(cmask, heat, _NEG_INF)
    scores_ref[...] = scores

    ri = lax.broadcasted_iota(jnp.int32, (_H, _W), 0)
    ci = lax.broadcasted_iota(jnp.int32, (_H, _W), 1)

    syv = ri.astype(jnp.float32) + off_ref[0]
    sxv = ci.astype(jnp.float32) + off_ref[1]
    sy_ref[...] = syv
    sx_ref[...] = sxv

    has_outlier = jnp.max(jnp.abs(off_ref[0])) > _MAX_OFF

    pan_ref[...] = sempred_ref[...]

    slabmax0 = jnp.max(scores.reshape(_N_SLABS, _SLAB, _W),
                       axis=(1, 2), keepdims=True)
    slab_io = lax.broadcasted_iota(jnp.int32, (_N_SLABS, 1, 1), 0)
    fi8 = (lax.broadcasted_iota(jnp.int32, (_SLAB, _W), 0) * _W
           + lax.broadcasted_iota(jnp.int32, (_SLAB, _W), 1))
    cio1 = lax.broadcasted_iota(jnp.int32, (1, _W), 1)
    big = jnp.int32(1 << 30)

    def body(k, carry):
        slabmax, next_id = carry
        m = jnp.max(slabmax)
        slab = jnp.min(jnp.where(slabmax == m, slab_io, big))
        sbase = pl.multiple_of(slab * _SLAB, _SLAB)
        vals = scores_ref[pl.ds(sbase, _SLAB), :]
        pos = jnp.min(jnp.where(vals == m, fi8, big))
        r_in = pos // _W
        col = pos % _W
        row = slab * _SLAB + r_in
        upd = jnp.where(fi8 == pos, _NEG_INF, vals)
        scores_ref[pl.ds(sbase, _SLAB), :] = upd
        newmax = jnp.max(upd)
        slabmax = jnp.where(slab_io == slab, newmax, slabmax)

        valid = (m > _NEG_INF) & (m < jnp.inf)

        semrow = sempred_ref[pl.ds(row, 1), :]
        cls = jnp.max(jnp.where(cio1 == col, semrow, jnp.int32(-1)))
        is_thing = (cls >= _THING_LO) & (cls <= _THING_HI)

        rowf = row.astype(jnp.float32)
        colf = col.astype(jnp.float32)

        wbase = (row // _SLAB) * _SLAB - 2 * _SLAB
        wstart = pl.multiple_of(
            jnp.clip(wbase, 0, _H - _WIN), _SLAB)
        syw = sy_ref[pl.ds(wstart, _WIN), :]
        sxw = sx_ref[pl.ds(wstart, _WIN), :]
        dyw = syw - rowf
        dxw = sxw - colf
        ind_w = dyw * dyw + dxw * dxw < 1.0
        cnt_s[0] = jnp.sum(ind_w.astype(jnp.int32))

        @pl.when(has_outlier)
        def _():
            dy = syv - rowf
            dx = sxv - colf
            ind_f = dy * dy + dx * dx < 1.0
            cnt_s[0] = jnp.sum(ind_f.astype(jnp.int32))

        accept = valid & (cnt_s[0] >= 32) & is_thing

        @pl.when(accept & jnp.logical_not(has_outlier))
        def _():
            panw = pan_ref[pl.ds(wstart, _WIN), :]
            pan_ref[pl.ds(wstart, _WIN), :] = jnp.where(ind_w, next_id, panw)

        @pl.when(accept & has_outlier)
        def _():
            dy = syv - rowf
            dx = sxv - colf
            ind_f = dy * dy + dx * dx < 1.0
            pan_ref[...] = jnp.where(ind_f, next_id, pan_ref[...])

        return (slabmax, next_id + accept.astype(jnp.int32))

    lax.fori_loop(0, _TOP_K, body, (slabmax0, jnp.int32(1000)))


@jax.jit
def kernel(semantic_logits, center_heatmap, offset_map):
    sem = semantic_logits[0]
    heat = center_heatmap[0, 0]
    off = offset_map[0]

    n_blk = 8
    rows = _H // n_blk
    sempred = pl.pallas_call(
        _argmax_body,
        grid=(n_blk,),
        in_specs=[pl.BlockSpec((1, _NUM_CLASSES, rows, _W),
                               lambda i: (0, 0, i, 0))],
        out_specs=pl.BlockSpec((rows, _W), lambda i: (i, 0)),
        out_shape=jax.ShapeDtypeStruct((_H, _W), jnp.int32),
    )(sem[None])

    pan = pl.pallas_call(
        _main_body,
        out_shape=jax.ShapeDtypeStruct((_H, _W), jnp.int32),
        scratch_shapes=[
            pltpu.VMEM((_H, _W), jnp.float32),
            pltpu.VMEM((_H, _W), jnp.float32),
            pltpu.VMEM((_H, _W), jnp.float32),
            pltpu.SMEM((1,), jnp.int32),
        ],
    )(heat, off, sempred)

    return (semantic_logits, center_heatmap, offset_map, pan[None])


# speedup vs baseline: 41.3357x; 2.0758x over previous
import functools

import jax
import jax.numpy as jnp
from jax import lax
from jax.experimental import pallas as pl
from jax.experimental.pallas import tpu as pltpu
from jax.experimental.pallas import tpu_sc as plsc

_NUM_CLASSES = 19
_THING_LO = 11
_THING_HI = 18
_CENTER_THRESHOLD = 0.1
_NMS_PAD = 3
_TOP_K = 200
_H = 384
_W = 384
_P = _H * _W
_NEG_INF = float("-inf")
_SLAB = 8
_WIN = 40
_MAX_OFF = 14.0

_NC = 2
_NS = 16
_NW = _NC * _NS
_CHUNK = _P // _NW
_GROUPS = _CHUNK // 16
_SLICE = _P // _NS
_NROWS = 4 * _CHUNK // 128


def _sc_count_body(oy_hbm, ox_hbm, out_hbm, oyv, oxv, idxb, valb, zb, cnt_sh):
    c = lax.axis_index("c")
    s = lax.axis_index("s")
    base = (c * _NS + s) * _CHUNK

    def zloop(i, carry):
        zb[pl.ds(i * 16, 16)] = jnp.zeros((16,), jnp.int32)
        return carry

    lax.fori_loop(0, _SLICE // 16, zloop, 0)
    pltpu.sync_copy(zb, cnt_sh.at[pl.ds(s * _SLICE, _SLICE)])
    plsc.subcore_barrier()

    pltpu.sync_copy(oy_hbm.at[pl.ds(base, _CHUNK)], oyv)
    pltpu.sync_copy(ox_hbm.at[pl.ds(base, _CHUNK)], oxv)

    lane = lax.iota(jnp.int32, 16)

    def gloop(g, carry):
        pid0 = base + g * 16
        y0 = pid0 // _W
        x0 = pid0 - y0 * _W
        sy = y0.astype(jnp.float32) + oyv[pl.ds(g * 16, 16)]
        sx = (x0 + lane).astype(jnp.float32) + oxv[pl.ds(g * 16, 16)]
        fy = sy.astype(jnp.int32)
        fy = jnp.where(fy.astype(jnp.float32) > sy, fy - 1, fy)
        fx = sx.astype(jnp.int32)
        fx = jnp.where(fx.astype(jnp.float32) > sx, fx - 1, fx)
        off0 = g * 64
        r = off0 // 128
        c0 = off0 - r * 128
        for ii, (dy_, dx_) in enumerate(((0, 0), (0, 1), (1, 0), (1, 1))):
            iy = fy + dy_
            ix = fx + dx_
            inb = (iy >= 0) & (iy < _H) & (ix >= 0) & (ix < _W)
            dyf = sy - iy.astype(jnp.float32)
            dxf = sx - ix.astype(jnp.float32)
            ind = (dyf * dyf + dxf * dxf < 1.0) & inb
            val = jnp.where(ind, jnp.int32(1), jnp.int32(0))
            idx = (jnp.clip(iy, 0, _H - 1) * _W
                   + jnp.clip(ix, 0, _W - 1))
            idxb[r, pl.ds(c0 + ii * 16, 16)] = idx
            valb[r, pl.ds(c0 + ii * 16, 16)] = val
        return carry

    lax.fori_loop(0, _GROUPS, gloop, 0)

    def sloop(j, carry):
        pltpu.sync_copy(valb.at[j], cnt_sh.at[idxb.at[j]], add=True)
        return carry

    lax.fori_loop(0, _NROWS, sloop, 0)
    plsc.subcore_barrier()

    pltpu.sync_copy(cnt_sh.at[pl.ds(s * _SLICE, _SLICE)],
                    out_hbm.at[c, pl.ds(s * _SLICE, _SLICE)])


def _sc_count(oy, ox):
    run = functools.partial(
        pl.kernel,
        mesh=plsc.VectorSubcoreMesh(core_axis_name="c", subcore_axis_name="s",
                                    num_cores=_NC, num_subcores=_NS),
        out_type=jax.ShapeDtypeStruct((_NC, _P), jnp.int32),
        scratch_types=[
            pltpu.VMEM((_CHUNK,), jnp.float32),
            pltpu.VMEM((_CHUNK,), jnp.float32),
            pltpu.VMEM((_NROWS, 128), jnp.int32),
            pltpu.VMEM((_NROWS, 128), jnp.int32),
            pltpu.VMEM((_SLICE,), jnp.int32),
            pltpu.VMEM_SHARED((_P,), jnp.int32),
        ],
    )(_sc_count_body)
    return run(oy, ox)


def _argmax_body(sem_ref, out_ref):
    x = sem_ref[0]
    best = x[0]
    bidx = jnp.zeros(best.shape, jnp.int32)
    for c in range(1, _NUM_CLASSES):
        better = x[c] > best
        best = jnp.where(better, x[c], best)
        bidx = jnp.where(better, c, bidx)
    out_ref[...] = bidx


def _fuse_body(heat_ref, off_ref, sempred_ref, cntp_ref, pan_ref,
               ascores_ref, sy_ref, sx_ref):
    heat = heat_ref[...]

    ninf_rows = jnp.full((_NMS_PAD, _W), _NEG_INF, jnp.float32)
    hp = jnp.concatenate([ninf_rows, heat, ninf_rows], axis=0)
    rm = hp[0:_H, :]
    for d in range(1, 2 * _NMS_PAD + 1):
        rm = jnp.maximum(rm, hp[d:d + _H, :])
    ninf_cols = jnp.full((_H, _NMS_PAD), _NEG_INF, jnp.float32)
    cp = jnp.concatenate([ninf_cols, rm, ninf_cols], axis=1)
    nms = cp[:, 0:_W]
    for d in range(1, 2 * _NMS_PAD + 1):
        nms = jnp.maximum(nms, cp[:, d:d + _W])

    cmask = (heat > _CENTER_THRESHOLD) & (heat == nms)
    scores = jnp.where(cmask, heat, _NEG_INF)

    sempred = sempred_ref[...]
    cnt = cntp_ref[0] + cntp_ref[1]
    thing = (sempred >= _THING_LO) & (sempred <= _THING_HI)
    acc = cmask & (heat < jnp.inf) & (cnt >= 32) & thing
    nacc = jnp.sum(acc.astype(jnp.int32))

    pan_ref[...] = sempred

    @pl.when(nacc > 0)
    def _():
        ri = lax.broadcasted_iota(jnp.int32, (_H, _W), 0)
        ci = lax.broadcasted_iota(jnp.int32, (_H, _W), 1)
        fidx = ri * _W + ci
        sy_ref[...] = ri.astype(jnp.float32) + off_ref[0]
        sx_ref[...] = ci.astype(jnp.float32) + off_ref[1]
        ascores_ref[...] = jnp.where(acc, scores, _NEG_INF)
        has_outlier = jnp.max(jnp.abs(off_ref[0])) > _MAX_OFF
        big = jnp.int32(1 << 30)

        def cond(carry):
            return carry[0] == 1

        def body(carry):
            _, next_id = carry
            a = ascores_ref[...]
            m = jnp.max(a)
            idx = jnp.min(jnp.where(a == m, fidx, big))
            row = idx // _W
            col = idx % _W
            ascores_ref[...] = jnp.where(fidx == idx, _NEG_INF, a)
            scnt = (jnp.sum((scores > m).astype(jnp.int32))
                    + jnp.sum(((scores == m) & (fidx < idx))
                              .astype(jnp.int32)))
            ok = (m > _NEG_INF) & (scnt < _TOP_K)
            rowf = row.astype(jnp.float32)
            colf = col.astype(jnp.float32)

            @pl.when(ok & jnp.logical_not(has_outlier))
            def _():
                wbase = (row // _SLAB) * _SLAB - 2 * _SLAB
                wstart = pl.multiple_of(
                    jnp.clip(wbase, 0, _H - _WIN), _SLAB)
                dyw = sy_ref[pl.ds(wstart, _WIN), :] - rowf
                dxw = sx_ref[pl.ds(wstart, _WIN), :] - colf
                ind_w = dyw * dyw + dxw * dxw < 1.0
                panw = pan_ref[pl.ds(wstart, _WIN), :]
                pan_ref[pl.ds(wstart, _WIN), :] = (
                    jnp.where(ind_w, next_id, panw))

            @pl.when(ok & has_outlier)
            def _():
                dy = sy_ref[...] - rowf
                dx = sx_ref[...] - colf
                ind_f = dy * dy + dx * dx < 1.0
                pan_ref[...] = jnp.where(ind_f, next_id, pan_ref[...])

            return (ok.astype(jnp.int32), next_id + ok.astype(jnp.int32))

        lax.while_loop(cond, body, (jnp.int32(1), jnp.int32(1000)))


@jax.jit
def kernel(semantic_logits, center_heatmap, offset_map):
    sem = semantic_logits[0]
    heat = center_heatmap[0, 0]
    off = offset_map[0]

    partial_counts = _sc_count(off[0].reshape(_P), off[1].reshape(_P))
    cntp = partial_counts.reshape(_NC, _H, _W)

    n_blk = 8
    rows = _H // n_blk
    sempred = pl.pallas_call(
        _argmax_body,
        grid=(n_blk,),
        in_specs=[pl.BlockSpec((1, _NUM_CLASSES, rows, _W),
                               lambda i: (0, 0, i, 0))],
        out_specs=pl.BlockSpec((rows, _W), lambda i: (i, 0)),
        out_shape=jax.ShapeDtypeStruct((_H, _W), jnp.int32),
    )(sem[None])

    pan = pl.pallas_call(
        _fuse_body,
        out_shape=jax.ShapeDtypeStruct((_H, _W), jnp.int32),
        scratch_shapes=[
            pltpu.VMEM((_H, _W), jnp.float32),
            pltpu.VMEM((_H, _W), jnp.float32),
            pltpu.VMEM((_H, _W), jnp.float32),
        ],
    )(heat, off, sempred, cntp)

    return (semantic_logits, center_heatmap, offset_map, pan[None])


# speedup vs baseline: 42.8275x; 1.0361x over previous
import functools

import jax
import jax.numpy as jnp
from jax import lax
from jax.experimental import pallas as pl
from jax.experimental.pallas import tpu as pltpu
from jax.experimental.pallas import tpu_sc as plsc

_NUM_CLASSES = 19
_THING_LO = 11
_THING_HI = 18
_CENTER_THRESHOLD = 0.1
_NMS_PAD = 3
_TOP_K = 200
_H = 384
_W = 384
_P = _H * _W
_NEG_INF = float("-inf")
_SLAB = 8
_WIN = 40
_MAX_OFF = 14.0

_NC = 2
_NS = 16
_NW = _NC * _NS
_CHUNK = _P // _NW
_GROUPS = _CHUNK // 16
_SLICE = _P // _NS
_NROWS = 4 * _CHUNK // 128


def _sc_count_body(oy_hbm, ox_hbm, out_hbm, oyv, oxv, idxb, valb, zb, cnt_sh,
                   dma_sem):
    c = lax.axis_index("c")
    s = lax.axis_index("s")
    base = (c * _NS + s) * _CHUNK

    def zloop(i, carry):
        zb[pl.ds(i * 16, 16)] = jnp.zeros((16,), jnp.int32)
        return carry

    lax.fori_loop(0, _SLICE // 16, zloop, 0)
    pltpu.sync_copy(zb, cnt_sh.at[pl.ds(s * _SLICE, _SLICE)])
    plsc.subcore_barrier()

    pltpu.sync_copy(oy_hbm.at[pl.ds(base, _CHUNK)], oyv)
    pltpu.sync_copy(ox_hbm.at[pl.ds(base, _CHUNK)], oxv)

    lane = lax.iota(jnp.int32, 16)

    def do_group(g, half):
        pid0 = base + g * 16
        y0 = pid0 // _W
        x0 = pid0 - y0 * _W
        sy = y0.astype(jnp.float32) + oyv[pl.ds(g * 16, 16)]
        sx = (x0 + lane).astype(jnp.float32) + oxv[pl.ds(g * 16, 16)]
        fy = sy.astype(jnp.int32)
        fy = jnp.where(fy.astype(jnp.float32) > sy, fy - 1, fy)
        fx = sx.astype(jnp.int32)
        fx = jnp.where(fx.astype(jnp.float32) > sx, fx - 1, fx)
        r = g // 2
        c0 = half * 64
        for ii, (dy_, dx_) in enumerate(((0, 0), (0, 1), (1, 0), (1, 1))):
            iy = fy + dy_
            ix = fx + dx_
            inb = (iy >= 0) & (iy < _H) & (ix >= 0) & (ix < _W)
            dyf = sy - iy.astype(jnp.float32)
            dxf = sx - ix.astype(jnp.float32)
            ind = (dyf * dyf + dxf * dxf < 1.0) & inb
            val = jnp.where(ind, jnp.int32(1), jnp.int32(0))
            idx = (jnp.clip(iy, 0, _H - 1) * _W
                   + jnp.clip(ix, 0, _W - 1))
            idxb[r, pl.ds(c0 + ii * 16, 16)] = idx
            valb[r, pl.ds(c0 + ii * 16, 16)] = val

    def gloop(i, carry):
        do_group(i * 2, 0)
        do_group(i * 2 + 1, 1)
        return carry

    lax.fori_loop(0, _GROUPS // 2, gloop, 0)

    _BATCH = 16

    def sloop(rr, carry):
        for b in range(_BATCH):
            j = rr * _BATCH + b
            pltpu.async_copy(valb.at[j], cnt_sh.at[idxb.at[j]], dma_sem,
                             add=True)
        for b in range(_BATCH):
            j = rr * _BATCH + b
            pltpu.make_async_copy(valb.at[j], cnt_sh.at[idxb.at[j]],
                                  dma_sem).wait()
        return carry

    lax.fori_loop(0, _NROWS // _BATCH, sloop, 0)
    plsc.subcore_barrier()

    pltpu.sync_copy(cnt_sh.at[pl.ds(s * _SLICE, _SLICE)],
                    out_hbm.at[c, pl.ds(s * _SLICE, _SLICE)])


def _sc_count(oy, ox):
    run = functools.partial(
        pl.kernel,
        mesh=plsc.VectorSubcoreMesh(core_axis_name="c", subcore_axis_name="s",
                                    num_cores=_NC, num_subcores=_NS),
        out_type=jax.ShapeDtypeStruct((_NC, _P), jnp.int32),
        scratch_types=[
            pltpu.VMEM((_CHUNK,), jnp.float32),
            pltpu.VMEM((_CHUNK,), jnp.float32),
            pltpu.VMEM((_NROWS, 128), jnp.int32),
            pltpu.VMEM((_NROWS, 128), jnp.int32),
            pltpu.VMEM((_SLICE,), jnp.int32),
            pltpu.VMEM_SHARED((_P,), jnp.int32),
            pltpu.SemaphoreType.DMA,
        ],
    )(_sc_count_body)
    return run(oy, ox)


_ROWBLK = 48
_NBLK = _H // _ROWBLK


def _fuse_body(sem_hbm, heat_ref, off_ref, cntp_ref, pan_ref,
               sempred_ref, ascores_ref, sy_ref, sx_ref,
               semb0, semb1, dsem0, dsem1):
    bufs = (semb0, semb1)
    sems = (dsem0, dsem1)
    pltpu.make_async_copy(
        sem_hbm.at[:, pl.ds(0, _ROWBLK), :], semb0, dsem0).start()
    for blk in range(_NBLK):
        if blk + 1 < _NBLK:
            pltpu.make_async_copy(
                sem_hbm.at[:, pl.ds((blk + 1) * _ROWBLK, _ROWBLK), :],
                bufs[(blk + 1) % 2], sems[(blk + 1) % 2]).start()
        buf = bufs[blk % 2]
        pltpu.make_async_copy(
            sem_hbm.at[:, pl.ds(blk * _ROWBLK, _ROWBLK), :],
            buf, sems[blk % 2]).wait()
        x = buf[...]
        best = x[0]
        bidx = jnp.zeros(best.shape, jnp.int32)
        for c in range(1, _NUM_CLASSES):
            better = x[c] > best
            best = jnp.where(better, x[c], best)
            bidx = jnp.where(better, c, bidx)
        sempred_ref[pl.ds(blk * _ROWBLK, _ROWBLK), :] = bidx

    heat = heat_ref[...]

    ninf_rows = jnp.full((_NMS_PAD, _W), _NEG_INF, jnp.float32)
    hp = jnp.concatenate([ninf_rows, heat, ninf_rows], axis=0)
    rm = hp[0:_H, :]
    for d in range(1, 2 * _NMS_PAD + 1):
        rm = jnp.maximum(rm, hp[d:d + _H, :])
    ninf_cols = jnp.full((_H, _NMS_PAD), _NEG_INF, jnp.float32)
    cp = jnp.concatenate([ninf_cols, rm, ninf_cols], axis=1)
    nms = cp[:, 0:_W]
    for d in range(1, 2 * _NMS_PAD + 1):
        nms = jnp.maximum(nms, cp[:, d:d + _W])

    cmask = (heat > _CENTER_THRESHOLD) & (heat == nms)
    scores = jnp.where(cmask, heat, _NEG_INF)

    sempred = sempred_ref[...]
    cnt = cntp_ref[0] + cntp_ref[1]
    thing = (sempred >= _THING_LO) & (sempred <= _THING_HI)
    acc = cmask & (heat < jnp.inf) & (cnt >= 32) & thing
    nacc = jnp.sum(acc.astype(jnp.int32))

    pan_ref[...] = sempred

    @pl.when(nacc > 0)
    def _():
        ri = lax.broadcasted_iota(jnp.int32, (_H, _W), 0)
        ci = lax.broadcasted_iota(jnp.int32, (_H, _W), 1)
        fidx = ri * _W + ci
        sy_ref[...] = ri.astype(jnp.float32) + off_ref[0]
        sx_ref[...] = ci.astype(jnp.float32) + off_ref[1]
        ascores_ref[...] = jnp.where(acc, scores, _NEG_INF)
        has_outlier = jnp.max(jnp.abs(off_ref[0])) > _MAX_OFF
        big = jnp.int32(1 << 30)

        def cond(carry):
            return carry[0] == 1

        def body(carry):
            _, next_id = carry
            a = ascores_ref[...]
            m = jnp.max(a)
            idx = jnp.min(jnp.where(a == m, fidx, big))
            row = idx // _W
            col = idx % _W
            ascores_ref[...] = jnp.where(fidx == idx, _NEG_INF, a)
            scnt = (jnp.sum((scores > m).astype(jnp.int32))
                    + jnp.sum(((scores == m) & (fidx < idx))
                              .astype(jnp.int32)))
            ok = (m > _NEG_INF) & (scnt < _TOP_K)
            rowf = row.astype(jnp.float32)
            colf = col.astype(jnp.float32)

            @pl.when(ok & jnp.logical_not(has_outlier))
            def _():
                wbase = (row // _SLAB) * _SLAB - 2 * _SLAB
                wstart = pl.multiple_of(
                    jnp.clip(wbase, 0, _H - _WIN), _SLAB)
                dyw = sy_ref[pl.ds(wstart, _WIN), :] - rowf
                dxw = sx_ref[pl.ds(wstart, _WIN), :] - colf
                ind_w = dyw * dyw + dxw * dxw < 1.0
                panw = pan_ref[pl.ds(wstart, _WIN), :]
                pan_ref[pl.ds(wstart, _WIN), :] = (
                    jnp.where(ind_w, next_id, panw))

            @pl.when(ok & has_outlier)
            def _():
                dy = sy_ref[...] - rowf
                dx = sx_ref[...] - colf
                ind_f = dy * dy + dx * dx < 1.0
                pan_ref[...] = jnp.where(ind_f, next_id, pan_ref[...])

            return (ok.astype(jnp.int32), next_id + ok.astype(jnp.int32))

        lax.while_loop(cond, body, (jnp.int32(1), jnp.int32(1000)))


@jax.jit
def kernel(semantic_logits, center_heatmap, offset_map):
    sem = semantic_logits[0]
    heat = center_heatmap[0, 0]
    off = offset_map[0]

    partial_counts = _sc_count(off[0].reshape(_P), off[1].reshape(_P))
    cntp = partial_counts.reshape(_NC, _H, _W)

    pan = pl.pallas_call(
        _fuse_body,
        out_shape=jax.ShapeDtypeStruct((_H, _W), jnp.int32),
        in_specs=[
            pl.BlockSpec(memory_space=pl.ANY),
            pl.BlockSpec(memory_space=pltpu.VMEM),
            pl.BlockSpec(memory_space=pltpu.VMEM),
            pl.BlockSpec(memory_space=pltpu.VMEM),
        ],
        scratch_shapes=[
            pltpu.VMEM((_H, _W), jnp.int32),
            pltpu.VMEM((_H, _W), jnp.float32),
            pltpu.VMEM((_H, _W), jnp.float32),
            pltpu.VMEM((_H, _W), jnp.float32),
            pltpu.VMEM((_NUM_CLASSES, _ROWBLK, _W), jnp.float32),
            pltpu.VMEM((_NUM_CLASSES, _ROWBLK, _W), jnp.float32),
            pltpu.SemaphoreType.DMA,
            pltpu.SemaphoreType.DMA,
        ],
    )(sem, heat, off, cntp)

    return (semantic_logits, center_heatmap, offset_map, pan[None])
